# baseline (speedup 1.0000x reference)
import jax
import jax.numpy as jnp
from jax import lax
from jax.experimental import pallas as pl
from jax.experimental.pallas import tpu as pltpu
from jax.experimental.pallas import tpu_sc as plsc

_T = 8192
_NB = 512
_B = 8
_L = 16
_NTILES = 32
_EPT = _T // 16
_GROUPS = _EPT // _L
_JT = 8
_VSTR = 264
_CSTR = 24
_XSTR = 72
_XCH = 8


def _sc_body(x3_hbm, vals_hbm, rows_hbm, cols_hbm, b_hbm, out_hbm,
             x3l, vbufA, vbufB, xstage, contribA, contribB, craw, vidxbuf,
             eidx, indbuf, ind0b, bbuf, out3, semv0, semv1, semx, semi,
             semc0, semc1):
    c = lax.axis_index("c")
    s = lax.axis_index("s")
    wid = s
    semv = (semv0, semv1)
    vbufs = (vbufA, vbufB)
    semc = (semc0, semc1)
    contribs = (contribA, contribB)
    contrib = contribA

    iota = lax.iota(jnp.int32, _L)
    iota_v = iota * _VSTR
    iota_c = iota * _CSTR

    def vals_issue(g, sub):
        base_e = wid * _EPT + g * _L
        for r in range(_L):
            pltpu.async_copy(vals_hbm.at[pl.ds((base_e + r) * 256, 256)],
                             vbufs[sub].at[pl.ds(r * _VSTR, 256)], semv[sub])

    def vals_wait(sub):
        for r in range(_L):
            pltpu.make_async_copy(vals_hbm.at[pl.ds(0, 256)],
                                  vbufs[sub].at[pl.ds(0, 256)],
                                  semv[sub]).wait()

    prolog_scope = jax.named_scope("prolog")
    prolog_scope.__enter__()
    iota_e = iota * 256
    for k in range(_GROUPS):
        vec = iota_e + (wid * _EPT + k * _L) * 256
        eidx[k // 8, pl.ds((k % 8) * _L, _L)] = vec
    for h in range(4):
        pltpu.async_copy(rows_hbm.at[eidx.at[h]], indbuf.at[h], semi)
        pltpu.async_copy(cols_hbm.at[eidx.at[h]], indbuf.at[4 + h], semi)

    xch = (_NB * _XSTR) // _XCH
    for k in range(_XCH):
        pltpu.async_copy(x3_hbm.at[c, pl.ds(k * xch, xch)],
                         x3l.at[pl.ds(k * xch, xch)], semx)

    for ij in range(256):
        vidxbuf[pl.ds(ij * _L, _L)] = iota_v + ij

    pbias = jax.named_scope("pbias")
    pbias.__enter__()
    pltpu.sync_copy(b_hbm.at[pl.ds(s * 32 * _L, 32 * _L)], bbuf)
    for half in range(2):
        for blk in range(_L):
            bv = bbuf[pl.ds((half * _L + blk) * _L, _L)]
            for bb in range(_B):
                contrib[blk, pl.ds(bb * _L, _L)] = bv
        pltpu.sync_copy(contrib, out3.at[pl.ds(s * 32 + half * _L, _L)])

    pbias.__exit__(None, None, None)
    pind = jax.named_scope("pind")
    pind.__enter__()
    for k in range(8):
        pltpu.make_async_copy(rows_hbm.at[pl.ds(0, 128)],
                              indbuf.at[0], semi).wait()
    for k in range(_GROUPS):
        rv = indbuf[k // 8, pl.ds((k % 8) * _L, _L)]
        ind0b[k, :] = lax.shift_right_logical(rv, 4)

    zv = jnp.zeros((_L,), jnp.float32)
    for cb in (contribA, contribB):
        for blk in range(_L):
            for bb in range(_B):
                cb[blk, pl.ds(bb * _L, _L)] = zv

    pind.__exit__(None, None, None)
    vals_issue(jnp.int32(0), 0)
    vals_issue(jnp.int32(1), 1)
    pxw = jax.named_scope("pxwait")
    pxw.__enter__()
    for k in range(_XCH):
        pltpu.make_async_copy(x3_hbm.at[0, pl.ds(0, xch)],
                              x3l.at[pl.ds(0, xch)], semx).wait()
    pxw.__exit__(None, None, None)
    prolog_scope.__exit__(None, None, None)
    with jax.named_scope("barrier1"):
        plsc.subcore_barrier()

    pltpu.async_copy(contribA, out3.at[ind0b.at[0]], semc0, add=True)
    pltpu.async_copy(contribB, out3.at[ind0b.at[0]], semc1, add=True)

    def compute_group(g, sub):
        with jax.named_scope("vwait"):
            vals_wait(sub)
        vbuf = vbufs[sub]
        ind1v = indbuf[4 + g // 8, pl.ds((g % 8) * _L, _L)]
        rowb = lax.shift_right_logical(ind1v, 4) * _XSTR

        with jax.named_scope("xpose"):
            for i in range(_L):
                xvs = [plsc.load_gather(x3l, [rowb + (i * 4 + bb)])
                       for bb in range(4)]
                for bb in range(4):
                    xstage[pl.ds((i * 4 + bb) * _L, _L)] = xvs[bb]

        def jt_body(jt, inner):
            jbase = jt * _JT
            acc = [[None] * 4 for _ in range(_JT)]
            for i in range(_L):
                vv = [plsc.load_gather(
                          vbuf,
                          [vidxbuf[pl.ds((i * 16 + jbase + jp) * _L, _L)]])
                      for jp in range(_JT)]
                xv = [xstage[pl.ds((i * 4 + bb) * _L, _L)]
                      for bb in range(4)]
                for jp in range(_JT):
                    for bb in range(4):
                        prod = vv[jp] * xv[bb]
                        if i == 0:
                            acc[jp][bb] = prod
                        else:
                            acc[jp][bb] = acc[jp][bb] + prod
            for jp in range(_JT):
                for bb in range(4):
                    col = bb * 16 + jbase + jp
                    start = pl.multiple_of(col * _CSTR, 8)
                    craw[pl.ds(start, _L)] = acc[jp][bb]
            return inner

        with jax.named_scope("jtloop"):
            lax.fori_loop(0, _L // _JT, jt_body, 0)

        with jax.named_scope("scatwait"):
            pltpu.make_async_copy(contribs[sub], out3.at[ind0b.at[0]],
                                  semc[sub]).wait()
        with jax.named_scope("conv"):
            for l in range(_L):
                vs = [plsc.load_gather(craw, [iota_c + (c0 * _CSTR + l)])
                      for c0 in range(0, 64, _L)]
                for k in range(4):
                    contribs[sub][l, pl.ds(k * _L, _L)] = vs[k]

        gnext = jnp.minimum(g + 2, _GROUPS - 1)
        vals_issue(gnext, sub)

        with jax.named_scope("scat"):
            pltpu.async_copy(contribs[sub], out3.at[ind0b.at[g]],
                             semc[sub], add=True)

    def pair_body(p, carry):
        compute_group(2 * p, 0)
        compute_group(2 * p + 1, 1)
        return carry

    lax.fori_loop(0, _GROUPS // 2, pair_body, 0)

    vals_wait(0)
    vals_wait(1)
    pltpu.make_async_copy(contribA, out3.at[ind0b.at[0]], semc0).wait()
    pltpu.make_async_copy(contribB, out3.at[ind0b.at[0]], semc1).wait()

    with jax.named_scope("barrier2"):
        plsc.subcore_barrier()
    with jax.named_scope("outcopy"):
        pltpu.sync_copy(out3.at[pl.ds(s * 32, 32)],
                        out_hbm.at[c, pl.ds(s * 32, 32)])


_KERNEL = pl.kernel(
    _sc_body,
    out_type=jax.ShapeDtypeStruct((2, _NB, 128), jnp.float32),
    mesh=plsc.VectorSubcoreMesh(core_axis_name="c", subcore_axis_name="s"),
    compiler_params=pltpu.CompilerParams(needs_layout_passes=False),
    scratch_types=[
        pltpu.VMEM((_NB * _XSTR,), jnp.float32),
        pltpu.VMEM((_L * _VSTR,), jnp.float32),
        pltpu.VMEM((_L * _VSTR,), jnp.float32),
        pltpu.VMEM((_L * 4 * _L,), jnp.float32),
        pltpu.VMEM((_L, 128), jnp.float32),
        pltpu.VMEM((_L, 128), jnp.float32),
        pltpu.VMEM((128 * _CSTR,), jnp.float32),
        pltpu.VMEM((256 * _L,), jnp.int32),
        pltpu.VMEM((4, 128), jnp.int32),
        pltpu.VMEM((8, 128), jnp.int32),
        pltpu.VMEM((_GROUPS, _L), jnp.int32),
        pltpu.VMEM((32 * _L,), jnp.float32),
        pltpu.VMEM_SHARED((_NB, 128), jnp.float32),
        pltpu.SemaphoreType.DMA,
        pltpu.SemaphoreType.DMA,
        pltpu.SemaphoreType.DMA,
        pltpu.SemaphoreType.DMA,
        pltpu.SemaphoreType.DMA,
        pltpu.SemaphoreType.DMA,
    ],
)


@jax.jit
def kernel(x, values, b, rows, cols):
    x4 = x.reshape(2, 4, _NB, _L).transpose(0, 2, 3, 1)
    x4 = x4.reshape(2, _NB, 64)
    x4 = jnp.pad(x4, ((0, 0), (0, 0), (0, _XSTR - 64))).reshape(2, -1)

    outp = _KERNEL(x4, values, rows.astype(jnp.int32), cols.astype(jnp.int32), b)
    out = outp[:, :, :64].reshape(2, _NB, 4, _L).transpose(0, 2, 1, 3)
    out = out.reshape(_B, _NB * _L, 1)
    return out


# speedup vs baseline: 1.0899x; 1.0899x over previous
import jax
import jax.numpy as jnp
from jax import lax
from jax.experimental import pallas as pl
from jax.experimental.pallas import tpu as pltpu
from jax.experimental.pallas import tpu_sc as plsc

_T = 8192
_NB = 512
_B = 8
_L = 16
_NTILES = 32
_EPT = _T // _NTILES
_GROUPS = _EPT // _L
_JT = 4
_VSTR = 264
_CSTR = 24
_XSTR = 136
_XCH = 16


def _sc_body(x3_hbm, vals_hbm, rows_hbm, cols_hbm, b_hbm, out_hbm,
             x3l, vbufA, vbufB, xstage, contribA, contribB, craw, vidxbuf,
             eidx, indbuf, ind0b, bbuf, out3, semv0, semv1, semx, semi,
             semc0, semc1):
    c = lax.axis_index("c")
    s = lax.axis_index("s")
    wid = c * 16 + s
    semv = (semv0, semv1)
    vbufs = (vbufA, vbufB)
    semc = (semc0, semc1)
    contribs = (contribA, contribB)
    contrib = contribA

    iota = lax.iota(jnp.int32, _L)
    iota_v = iota * _VSTR
    iota_c = iota * _CSTR

    def vals_issue(g, sub):
        base_e = wid * _EPT + g * _L
        for r in range(_L):
            pltpu.async_copy(vals_hbm.at[pl.ds((base_e + r) * 256, 256)],
                             vbufs[sub].at[pl.ds(r * _VSTR, 256)], semv[sub])

    def vals_wait(sub):
        for r in range(_L):
            pltpu.make_async_copy(vals_hbm.at[pl.ds(0, 256)],
                                  vbufs[sub].at[pl.ds(0, 256)],
                                  semv[sub]).wait()

    prolog_scope = jax.named_scope("prolog")
    prolog_scope.__enter__()
    iota_e = iota * 256
    for k in range(_GROUPS):
        vec = iota_e + (wid * _EPT + k * _L) * 256
        eidx[k // 8, pl.ds((k % 8) * _L, _L)] = vec
    pltpu.async_copy(rows_hbm.at[eidx.at[0]], indbuf.at[0], semi)
    pltpu.async_copy(rows_hbm.at[eidx.at[1]], indbuf.at[1], semi)
    pltpu.async_copy(cols_hbm.at[eidx.at[0]], indbuf.at[2], semi)
    pltpu.async_copy(cols_hbm.at[eidx.at[1]], indbuf.at[3], semi)

    xch = (_NB * _XSTR) // _XCH
    for k in range(_XCH):
        pltpu.async_copy(x3_hbm.at[pl.ds(k * xch, xch)],
                         x3l.at[pl.ds(k * xch, xch)], semx)

    for ij in range(256):
        vidxbuf[pl.ds(ij * _L, _L)] = iota_v + ij

    pbias = jax.named_scope("pbias")
    pbias.__enter__()
    pltpu.sync_copy(b_hbm.at[pl.ds(s * 32 * _L, 32 * _L)], bbuf)
    zero = jnp.zeros((_L,), jnp.float32)
    for half in range(2):
        for blk in range(_L):
            bv = bbuf[pl.ds((half * _L + blk) * _L, _L)]
            bv = jnp.where(c == 0, bv, zero)
            for bb in range(_B):
                contrib[blk, pl.ds(bb * _L, _L)] = bv
        pltpu.sync_copy(contrib, out3.at[pl.ds(s * 32 + half * _L, _L)])

    pbias.__exit__(None, None, None)
    pind = jax.named_scope("pind")
    pind.__enter__()
    for k in range(4):
        pltpu.make_async_copy(rows_hbm.at[pl.ds(0, 128)],
                              indbuf.at[0], semi).wait()
    for k in range(_GROUPS):
        rv = indbuf[k // 8, pl.ds((k % 8) * _L, _L)]
        ind0b[k, :] = lax.shift_right_logical(rv, 4)

    zv = jnp.zeros((_L,), jnp.float32)
    for cb in (contribA, contribB):
        for blk in range(_L):
            for bb in range(_B):
                cb[blk, pl.ds(bb * _L, _L)] = zv

    pind.__exit__(None, None, None)
    vals_issue(jnp.int32(0), 0)
    vals_issue(jnp.int32(1), 1)
    pxw = jax.named_scope("pxwait")
    pxw.__enter__()
    for k in range(_XCH):
        pltpu.make_async_copy(x3_hbm.at[pl.ds(0, xch)],
                              x3l.at[pl.ds(0, xch)], semx).wait()
    pxw.__exit__(None, None, None)
    prolog_scope.__exit__(None, None, None)
    with jax.named_scope("barrier1"):
        plsc.subcore_barrier()

    pltpu.async_copy(contribA, out3.at[ind0b.at[0]], semc0, add=True)
    pltpu.async_copy(contribB, out3.at[ind0b.at[0]], semc1, add=True)

    def compute_group(g, sub):
        with jax.named_scope("vwait"):
            vals_wait(sub)
        vbuf = vbufs[sub]
        ind1v = indbuf[2 + g // 8, pl.ds((g % 8) * _L, _L)]
        rowb = lax.shift_right_logical(ind1v, 4) * _XSTR

        with jax.named_scope("xpose"):
            for i in range(_L):
                xvs = [plsc.load_gather(x3l, [rowb + (i * _B + bb)])
                       for bb in range(_B)]
                for bb in range(_B):
                    xstage[pl.ds((i * _B + bb) * _L, _L)] = xvs[bb]

        def jt_body(jt, inner):
            jbase = jt * _JT
            acc = [[None] * _B for _ in range(_JT)]
            for i in range(_L):
                vv = [plsc.load_gather(
                          vbuf,
                          [vidxbuf[pl.ds((i * 16 + jbase + jp) * _L, _L)]])
                      for jp in range(_JT)]
                xv = [xstage[pl.ds((i * _B + bb) * _L, _L)]
                      for bb in range(_B)]
                for jp in range(_JT):
                    for bb in range(_B):
                        prod = vv[jp] * xv[bb]
                        if i == 0:
                            acc[jp][bb] = prod
                        else:
                            acc[jp][bb] = acc[jp][bb] + prod
            for jp in range(_JT):
                for bb in range(_B):
                    col = bb * 16 + jbase + jp
                    start = pl.multiple_of(col * _CSTR, 8)
                    craw[pl.ds(start, _L)] = acc[jp][bb]
            return inner

        with jax.named_scope("jtloop"):
            lax.fori_loop(0, _L // _JT, jt_body, 0)

        with jax.named_scope("scatwait"):
            pltpu.make_async_copy(contribs[sub], out3.at[ind0b.at[0]],
                                  semc[sub]).wait()
        with jax.named_scope("conv"):
            for l in range(_L):
                vs = [plsc.load_gather(craw, [iota_c + (c0 * _CSTR + l)])
                      for c0 in range(0, 128, _L)]
                for k in range(_B):
                    contribs[sub][l, pl.ds(k * _L, _L)] = vs[k]

        gnext = jnp.minimum(g + 2, _GROUPS - 1)
        vals_issue(gnext, sub)

        with jax.named_scope("scat"):
            pltpu.async_copy(contribs[sub], out3.at[ind0b.at[g]],
                             semc[sub], add=True)

    def pair_body(p, carry):
        compute_group(2 * p, 0)
        compute_group(2 * p + 1, 1)
        return carry

    lax.fori_loop(0, _GROUPS // 2, pair_body, 0)

    vals_wait(0)
    vals_wait(1)
    pltpu.make_async_copy(contribA, out3.at[ind0b.at[0]], semc0).wait()
    pltpu.make_async_copy(contribB, out3.at[ind0b.at[0]], semc1).wait()

    with jax.named_scope("barrier2"):
        plsc.subcore_barrier()
    with jax.named_scope("outcopy"):
        pltpu.sync_copy(out3.at[pl.ds(s * 32, 32)],
                        out_hbm.at[c, pl.ds(s * 32, 32)])


_KERNEL = pl.kernel(
    _sc_body,
    out_type=jax.ShapeDtypeStruct((2, _NB, 128), jnp.float32),
    mesh=plsc.VectorSubcoreMesh(core_axis_name="c", subcore_axis_name="s"),
    compiler_params=pltpu.CompilerParams(needs_layout_passes=False),
    scratch_types=[
        pltpu.VMEM((_NB * _XSTR,), jnp.float32),
        pltpu.VMEM((_L * _VSTR,), jnp.float32),
        pltpu.VMEM((_L * _VSTR,), jnp.float32),
        pltpu.VMEM((_L * _B * _L,), jnp.float32),
        pltpu.VMEM((_L, 128), jnp.float32),
        pltpu.VMEM((_L, 128), jnp.float32),
        pltpu.VMEM((128 * _CSTR,), jnp.float32),
        pltpu.VMEM((256 * _L,), jnp.int32),
        pltpu.VMEM((2, 128), jnp.int32),
        pltpu.VMEM((4, 128), jnp.int32),
        pltpu.VMEM((_GROUPS, _L), jnp.int32),
        pltpu.VMEM((32 * _L,), jnp.float32),
        pltpu.VMEM_SHARED((_NB, 128), jnp.float32),
        pltpu.SemaphoreType.DMA,
        pltpu.SemaphoreType.DMA,
        pltpu.SemaphoreType.DMA,
        pltpu.SemaphoreType.DMA,
        pltpu.SemaphoreType.DMA,
        pltpu.SemaphoreType.DMA,
    ],
)


@jax.jit
def kernel(x, values, b, rows, cols):
    x3 = x.reshape(_B, _NB, _L).transpose(1, 2, 0).reshape(_NB, 128)
    x3 = jnp.pad(x3, ((0, 0), (0, _XSTR - 128))).reshape(-1)

    outp = _KERNEL(x3, values, rows.astype(jnp.int32), cols.astype(jnp.int32), b)
    out = outp[0] + outp[1]
    out = out.reshape(_NB, _B, _L).transpose(1, 0, 2).reshape(_B, _NB * _L, 1)
    return out


# speedup vs baseline: 1.0989x; 1.0082x over previous
import jax
import jax.numpy as jnp
from jax import lax
from jax.experimental import pallas as pl
from jax.experimental.pallas import tpu as pltpu
from jax.experimental.pallas import tpu_sc as plsc

_T = 8192
_NB = 512
_B = 8
_L = 16
_NTILES = 32
_EPT = _T // _NTILES
_GROUPS = _EPT // _L
_JT = 4
_VSTR = 264
_CSTR = 24
_XSTR = 136
_XCH = 8


def _sc_body(x3_hbm, vals_hbm, rows_hbm, cols_hbm, b_hbm, out_hbm,
             x3l, vbufA, vbufB, xstage, contribA, contribB, craw, vidxbuf,
             eidx, indbuf, ind0b, bbuf, out3, semv0, semv1, semx, semi,
             semc0, semc1):
    c = lax.axis_index("c")
    s = lax.axis_index("s")
    wid = c * 16 + s
    semv = (semv0, semv1)
    vbufs = (vbufA, vbufB)
    semc = (semc0, semc1)
    contribs = (contribA, contribB)
    contrib = contribA

    iota = lax.iota(jnp.int32, _L)
    iota_v = iota * _VSTR
    iota_c = iota * _CSTR

    def vals_issue(g, sub):
        base_e = wid * _EPT + g * _L
        for r in range(_L):
            pltpu.async_copy(vals_hbm.at[pl.ds((base_e + r) * 256, 256)],
                             vbufs[sub].at[pl.ds(r * _VSTR, 256)], semv[sub])

    def vals_wait(sub):
        for r in range(_L):
            pltpu.make_async_copy(vals_hbm.at[pl.ds(0, 256)],
                                  vbufs[sub].at[pl.ds(0, 256)],
                                  semv[sub]).wait()

    prolog_scope = jax.named_scope("prolog")
    prolog_scope.__enter__()
    iota_e = iota * 256
    for k in range(_GROUPS):
        vec = iota_e + (wid * _EPT + k * _L) * 256
        eidx[k // 8, pl.ds((k % 8) * _L, _L)] = vec
    pltpu.async_copy(rows_hbm.at[eidx.at[0]], indbuf.at[0], semi)
    pltpu.async_copy(rows_hbm.at[eidx.at[1]], indbuf.at[1], semi)
    pltpu.async_copy(cols_hbm.at[eidx.at[0]], indbuf.at[2], semi)
    pltpu.async_copy(cols_hbm.at[eidx.at[1]], indbuf.at[3], semi)

    xch = (_NB * _XSTR) // _XCH
    for k in range(_XCH):
        pltpu.async_copy(x3_hbm.at[pl.ds(k * xch, xch)],
                         x3l.at[pl.ds(k * xch, xch)], semx)

    for ij in range(256):
        vidxbuf[pl.ds(ij * _L, _L)] = iota_v + ij

    pbias = jax.named_scope("pbias")
    pbias.__enter__()
    pltpu.sync_copy(b_hbm.at[pl.ds(s * 32 * _L, 32 * _L)], bbuf)
    zero = jnp.zeros((_L,), jnp.float32)
    for half in range(2):
        for blk in range(_L):
            bv = bbuf[pl.ds((half * _L + blk) * _L, _L)]
            bv = jnp.where(c == 0, bv, zero)
            for bb in range(_B):
                contrib[blk, pl.ds(bb * _L, _L)] = bv
        pltpu.sync_copy(contrib, out3.at[pl.ds(s * 32 + half * _L, _L)])

    pbias.__exit__(None, None, None)
    pind = jax.named_scope("pind")
    pind.__enter__()
    for k in range(4):
        pltpu.make_async_copy(rows_hbm.at[pl.ds(0, 128)],
                              indbuf.at[0], semi).wait()
    for k in range(_GROUPS):
        rv = indbuf[k // 8, pl.ds((k % 8) * _L, _L)]
        ind0b[k, :] = lax.shift_right_logical(rv, 4)

    zv = jnp.zeros((_L,), jnp.float32)
    for cb in (contribA, contribB):
        for blk in range(_L):
            for bb in range(_B):
                cb[blk, pl.ds(bb * _L, _L)] = zv

    pind.__exit__(None, None, None)
    vals_issue(jnp.int32(0), 0)
    vals_issue(jnp.int32(1), 1)
    pxw = jax.named_scope("pxwait")
    pxw.__enter__()
    for k in range(_XCH):
        pltpu.make_async_copy(x3_hbm.at[pl.ds(0, xch)],
                              x3l.at[pl.ds(0, xch)], semx).wait()
    pxw.__exit__(None, None, None)
    prolog_scope.__exit__(None, None, None)
    with jax.named_scope("barrier1"):
        plsc.subcore_barrier()

    pltpu.async_copy(contribA, out3.at[ind0b.at[0]], semc0, add=True)
    pltpu.async_copy(contribB, out3.at[ind0b.at[0]], semc1, add=True)

    def compute_group(g, sub):
        with jax.named_scope("vwait"):
            vals_wait(sub)
        vbuf = vbufs[sub]
        ind1v = indbuf[2 + g // 8, pl.ds((g % 8) * _L, _L)]
        rowb = lax.shift_right_logical(ind1v, 4) * _XSTR

        with jax.named_scope("xpose"):
            for i in range(_L):
                xvs = [plsc.load_gather(x3l, [rowb + (i * _B + bb)])
                       for bb in range(_B)]
                for bb in range(_B):
                    xstage[pl.ds((i * _B + bb) * _L, _L)] = xvs[bb]

        def jt_body(jt, inner):
            jbase = jt * _JT
            acc = [[None] * _B for _ in range(_JT)]
            for i in range(_L):
                vv = [plsc.load_gather(
                          vbuf,
                          [vidxbuf[pl.ds((i * 16 + jbase + jp) * _L, _L)]])
                      for jp in range(_JT)]
                xv = [xstage[pl.ds((i * _B + bb) * _L, _L)]
                      for bb in range(_B)]
                for jp in range(_JT):
                    for bb in range(_B):
                        prod = vv[jp] * xv[bb]
                        if i == 0:
                            acc[jp][bb] = prod
                        else:
                            acc[jp][bb] = acc[jp][bb] + prod
            for jp in range(_JT):
                for bb in range(_B):
                    col = bb * 16 + jbase + jp
                    start = pl.multiple_of(col * _CSTR, 8)
                    craw[pl.ds(start, _L)] = acc[jp][bb]
            return inner

        with jax.named_scope("jtloop"):
            lax.fori_loop(0, _L // _JT, jt_body, 0)

        with jax.named_scope("scatwait"):
            pltpu.make_async_copy(contribs[sub], out3.at[ind0b.at[0]],
                                  semc[sub]).wait()
        with jax.named_scope("conv"):
            for l in range(_L):
                vs = [plsc.load_gather(craw, [iota_c + (c0 * _CSTR + l)])
                      for c0 in range(0, 128, _L)]
                for k in range(_B):
                    contribs[sub][l, pl.ds(k * _L, _L)] = vs[k]

        gnext = jnp.minimum(g + 2, _GROUPS - 1)
        vals_issue(gnext, sub)

        with jax.named_scope("scat"):
            pltpu.async_copy(contribs[sub], out3.at[ind0b.at[g]],
                             semc[sub], add=True)

    def pair_body(p, carry):
        compute_group(2 * p, 0)
        compute_group(2 * p + 1, 1)
        return carry

    lax.fori_loop(0, _GROUPS // 2, pair_body, 0)

    vals_wait(0)
    vals_wait(1)
    pltpu.make_async_copy(contribA, out3.at[ind0b.at[0]], semc0).wait()
    pltpu.make_async_copy(contribB, out3.at[ind0b.at[0]], semc1).wait()

    with jax.named_scope("barrier2"):
        plsc.subcore_barrier()
    with jax.named_scope("outcopy"):
        pltpu.sync_copy(out3.at[pl.ds(s * 32, 32)],
                        out_hbm.at[c, pl.ds(s * 32, 32)])


_KERNEL = pl.kernel(
    _sc_body,
    out_type=jax.ShapeDtypeStruct((2, _NB, 128), jnp.float32),
    mesh=plsc.VectorSubcoreMesh(core_axis_name="c", subcore_axis_name="s"),
    compiler_params=pltpu.CompilerParams(needs_layout_passes=False),
    scratch_types=[
        pltpu.VMEM((_NB * _XSTR,), jnp.float32),
        pltpu.VMEM((_L * _VSTR,), jnp.float32),
        pltpu.VMEM((_L * _VSTR,), jnp.float32),
        pltpu.VMEM((_L * _B * _L,), jnp.float32),
        pltpu.VMEM((_L, 128), jnp.float32),
        pltpu.VMEM((_L, 128), jnp.float32),
        pltpu.VMEM((128 * _CSTR,), jnp.float32),
        pltpu.VMEM((256 * _L,), jnp.int32),
        pltpu.VMEM((2, 128), jnp.int32),
        pltpu.VMEM((4, 128), jnp.int32),
        pltpu.VMEM((_GROUPS, _L), jnp.int32),
        pltpu.VMEM((32 * _L,), jnp.float32),
        pltpu.VMEM_SHARED((_NB, 128), jnp.float32),
        pltpu.SemaphoreType.DMA,
        pltpu.SemaphoreType.DMA,
        pltpu.SemaphoreType.DMA,
        pltpu.SemaphoreType.DMA,
        pltpu.SemaphoreType.DMA,
        pltpu.SemaphoreType.DMA,
    ],
)


@jax.jit
def kernel(x, values, b, rows, cols):
    x3 = x.reshape(_B, _NB, _L).transpose(1, 2, 0).reshape(_NB, 128)
    x3 = jnp.pad(x3, ((0, 0), (0, _XSTR - 128))).reshape(-1)

    outp = _KERNEL(x3, values, rows.astype(jnp.int32), cols.astype(jnp.int32), b)
    out = outp[0] + outp[1]
    out = out.reshape(_NB, _B, _L).transpose(1, 0, 2).reshape(_B, _NB * _L, 1)
    return out


# speedup vs baseline: 1.1172x; 1.0167x over previous
import jax
import jax.numpy as jnp
from jax import lax
from jax.experimental import pallas as pl
from jax.experimental.pallas import tpu as pltpu
from jax.experimental.pallas import tpu_sc as plsc

_T = 8192
_NB = 512
_B = 8
_L = 16
_NTILES = 32
_EPT = _T // _NTILES
_GROUPS = _EPT // _L
_JT = 4
_VSTR = 264
_CSTR = 24
_XSTR = 136
_XCH = 8


def _sc_body(x3_hbm, vals_hbm, rows_hbm, cols_hbm, b_hbm, out_hbm,
             x3l, vbufA, vbufB, xstage, contribA, contribB, craw, vidxbuf,
             eidx, indbuf, ind0b, bbuf, out3, semv0, semv1, semx, semi,
             semc0, semc1):
    c = lax.axis_index("c")
    s = lax.axis_index("s")
    wid = c * 16 + s
    semv = (semv0, semv1)
    vbufs = (vbufA, vbufB)
    semc = (semc0, semc1)
    contribs = (contribA, contribB)
    contrib = contribA

    iota = lax.iota(jnp.int32, _L)
    iota_v = iota * _VSTR
    iota_c = iota * _CSTR

    def vals_issue(g, sub):
        base_e = wid * _EPT + g * _L
        for r in range(_L):
            pltpu.async_copy(vals_hbm.at[pl.ds((base_e + r) * 256, 256)],
                             vbufs[sub].at[pl.ds(r * _VSTR, 256)], semv[sub])

    def vals_wait(sub):
        for r in range(_L):
            pltpu.make_async_copy(vals_hbm.at[pl.ds(0, 256)],
                                  vbufs[sub].at[pl.ds(0, 256)],
                                  semv[sub]).wait()

    prolog_scope = jax.named_scope("prolog")
    prolog_scope.__enter__()
    iota_e = iota * 256
    for k in range(_GROUPS):
        vec = iota_e + (wid * _EPT + k * _L) * 256
        eidx[k // 8, pl.ds((k % 8) * _L, _L)] = vec
    pltpu.async_copy(rows_hbm.at[eidx.at[0]], indbuf.at[0], semi)
    pltpu.async_copy(rows_hbm.at[eidx.at[1]], indbuf.at[1], semi)
    pltpu.async_copy(cols_hbm.at[eidx.at[0]], indbuf.at[2], semi)
    pltpu.async_copy(cols_hbm.at[eidx.at[1]], indbuf.at[3], semi)

    xch = (_NB * _XSTR) // _XCH
    for k in range(_XCH):
        pltpu.async_copy(x3_hbm.at[pl.ds(k * xch, xch)],
                         x3l.at[pl.ds(k * xch, xch)], semx)

    for ij in range(256):
        vidxbuf[pl.ds(ij * _L, _L)] = iota_v + ij

    pbias = jax.named_scope("pbias")
    pbias.__enter__()
    pltpu.sync_copy(b_hbm.at[pl.ds(s * 32 * _L, 32 * _L)], bbuf)
    zero = jnp.zeros((_L,), jnp.float32)
    for half in range(2):
        for blk in range(_L):
            bv = bbuf[pl.ds((half * _L + blk) * _L, _L)]
            bv = jnp.where(c == 0, bv, zero)
            for bb in range(_B):
                contrib[blk, pl.ds(bb * _L, _L)] = bv
        pltpu.sync_copy(contrib, out3.at[pl.ds(s * 32 + half * _L, _L)])

    pbias.__exit__(None, None, None)
    pind = jax.named_scope("pind")
    pind.__enter__()
    for k in range(4):
        pltpu.make_async_copy(rows_hbm.at[pl.ds(0, 128)],
                              indbuf.at[0], semi).wait()
    for k in range(_GROUPS):
        rv = indbuf[k // 8, pl.ds((k % 8) * _L, _L)]
        ind0b[k, :] = lax.shift_right_logical(rv, 4)

    zv = jnp.zeros((_L,), jnp.float32)
    for cb in (contribA, contribB):
        for blk in range(_L):
            for bb in range(_B):
                cb[blk, pl.ds(bb * _L, _L)] = zv

    pind.__exit__(None, None, None)
    vals_issue(jnp.int32(0), 0)
    vals_issue(jnp.int32(1), 1)
    pxw = jax.named_scope("pxwait")
    pxw.__enter__()
    for k in range(_XCH):
        pltpu.make_async_copy(x3_hbm.at[pl.ds(0, xch)],
                              x3l.at[pl.ds(0, xch)], semx).wait()
    pxw.__exit__(None, None, None)
    prolog_scope.__exit__(None, None, None)
    with jax.named_scope("barrier1"):
        plsc.subcore_barrier()

    pltpu.async_copy(contribA, out3.at[ind0b.at[0]], semc0, add=True)
    pltpu.async_copy(contribB, out3.at[ind0b.at[0]], semc1, add=True)

    def compute_group(g, sub):
        with jax.named_scope("vwait"):
            vals_wait(sub)
        vbuf = vbufs[sub]
        ind1v = indbuf[2 + g // 8, pl.ds((g % 8) * _L, _L)]
        rowb = lax.shift_right_logical(ind1v, 4) * _XSTR

        with jax.named_scope("xpose"):
            for i in range(_L):
                xvs = [plsc.load_gather(x3l, [rowb + (i * _B + bb)])
                       for bb in range(_B)]
                for bb in range(_B):
                    xstage[pl.ds((i * _B + bb) * _L, _L)] = xvs[bb]

        def jt_body(jt, inner):
            jbase = jt * _JT
            acc = [[None] * _B for _ in range(_JT)]
            for i in range(_L):
                vv = [plsc.load_gather(
                          vbuf,
                          [vidxbuf[pl.ds((i * 16 + jbase + jp) * _L, _L)]])
                      for jp in range(_JT)]
                xv = [xstage[pl.ds((i * _B + bb) * _L, _L)]
                      for bb in range(_B)]
                for jp in range(_JT):
                    for bb in range(_B):
                        prod = vv[jp] * xv[bb]
                        if i == 0:
                            acc[jp][bb] = prod
                        else:
                            acc[jp][bb] = acc[jp][bb] + prod
            for jp in range(_JT):
                for bb in range(_B):
                    col = bb * 16 + jbase + jp
                    start = pl.multiple_of(col * _CSTR, 8)
                    craw[pl.ds(start, _L)] = acc[jp][bb]
            return inner

        with jax.named_scope("jtloop"):
            lax.fori_loop(0, _L // _JT, jt_body, 0)

        with jax.named_scope("scatwait"):
            pltpu.make_async_copy(contribs[sub], out3.at[ind0b.at[0]],
                                  semc[sub]).wait()
        with jax.named_scope("conv"):
            for l in range(_L):
                vs = [plsc.load_gather(craw, [iota_c + (c0 * _CSTR + l)])
                      for c0 in range(0, 128, _L)]
                for k in range(_B):
                    contribs[sub][l, pl.ds(k * _L, _L)] = vs[k]

        @pl.when(g + 2 < _GROUPS)
        def _():
            vals_issue(g + 2, sub)

        with jax.named_scope("scat"):
            pltpu.async_copy(contribs[sub], out3.at[ind0b.at[g]],
                             semc[sub], add=True)

    def pair_body(p, carry):
        compute_group(2 * p, 0)
        compute_group(2 * p + 1, 1)
        return carry

    lax.fori_loop(0, _GROUPS // 2, pair_body, 0)

    pltpu.make_async_copy(contribA, out3.at[ind0b.at[0]], semc0).wait()
    pltpu.make_async_copy(contribB, out3.at[ind0b.at[0]], semc1).wait()

    with jax.named_scope("barrier2"):
        plsc.subcore_barrier()
    with jax.named_scope("outcopy"):
        pltpu.sync_copy(out3.at[pl.ds(s * 32, 32)],
                        out_hbm.at[c, pl.ds(s * 32, 32)])


_KERNEL = pl.kernel(
    _sc_body,
    out_type=jax.ShapeDtypeStruct((2, _NB, 128), jnp.float32),
    mesh=plsc.VectorSubcoreMesh(core_axis_name="c", subcore_axis_name="s"),
    compiler_params=pltpu.CompilerParams(needs_layout_passes=False),
    scratch_types=[
        pltpu.VMEM((_NB * _XSTR,), jnp.float32),
        pltpu.VMEM((_L * _VSTR,), jnp.float32),
        pltpu.VMEM((_L * _VSTR,), jnp.float32),
        pltpu.VMEM((_L * _B * _L,), jnp.float32),
        pltpu.VMEM((_L, 128), jnp.float32),
        pltpu.VMEM((_L, 128), jnp.float32),
        pltpu.VMEM((128 * _CSTR,), jnp.float32),
        pltpu.VMEM((256 * _L,), jnp.int32),
        pltpu.VMEM((2, 128), jnp.int32),
        pltpu.VMEM((4, 128), jnp.int32),
        pltpu.VMEM((_GROUPS, _L), jnp.int32),
        pltpu.VMEM((32 * _L,), jnp.float32),
        pltpu.VMEM_SHARED((_NB, 128), jnp.float32),
        pltpu.SemaphoreType.DMA,
        pltpu.SemaphoreType.DMA,
        pltpu.SemaphoreType.DMA,
        pltpu.SemaphoreType.DMA,
        pltpu.SemaphoreType.DMA,
        pltpu.SemaphoreType.DMA,
    ],
)


@jax.jit
def kernel(x, values, b, rows, cols):
    x3 = x.reshape(_B, _NB, _L).transpose(1, 2, 0).reshape(_NB, 128)
    x3 = jnp.pad(x3, ((0, 0), (0, _XSTR - 128))).reshape(-1)

    outp = _KERNEL(x3, values, rows.astype(jnp.int32), cols.astype(jnp.int32), b)
    out = outp[0] + outp[1]
    out = out.reshape(_NB, _B, _L).transpose(1, 0, 2).reshape(_B, _NB * _L, 1)
    return out
